# baseline (speedup 1.0000x reference)
import functools

import jax
import jax.numpy as jnp
from jax import lax
from jax.experimental import pallas as pl
from jax.experimental.pallas import tpu as pltpu
from jax.experimental.pallas import tpu_sc as plsc


def _build_kernel(B, L, V, D):
    info = plsc.get_sparse_core_info()
    NC, NS, NL = info.num_cores, info.num_subcores, info.num_lanes
    NW = NC * NS
    assert B % NW == 0
    SPW = B // NW
    C = 16
    assert SPW % C == 0
    NCH = SPW // C
    CL = C * L
    assert D % NL == 0 and D % 8 == 0
    DG = D // NL
    DT = D // 8
    BTn = B // 128
    pieces = []
    off = 0
    while off < CL:
        n = min(128, CL - off)
        pieces.append((off, n))
        off += n

    mesh = plsc.VectorSubcoreMesh(core_axis_name="c", subcore_axis_name="s")

    @functools.partial(
        pl.kernel,
        mesh=mesh,
        compiler_params=pltpu.CompilerParams(
            use_tc_tiling_on_sc=False, needs_layout_passes=False
        ),
        out_type=(
            jax.ShapeDtypeStruct((L, DT, BTn, 8, 128), jnp.float32),
            jax.ShapeDtypeStruct((DT, BTn, 8, 128), jnp.float32),
        ),
        scratch_types=[
            pltpu.VMEM((CL,), jnp.int32),
            pltpu.VMEM((CL, D), jnp.float32),
            pltpu.VMEM((SPW,), jnp.int32),
            pltpu.VMEM((L, DT, 8, C), jnp.float32),
            pltpu.VMEM((C * D,), jnp.float32),
            pltpu.VMEM((DT, 8, C), jnp.float32),
            pltpu.SemaphoreType.DMA,
            pltpu.SemaphoreType.DMA,
        ],
    )
    def sc_kernel(text_ref, len_ref, table_ref, embs_ref, semb_ref,
                  idx_v, rows_v, lens_v, t5_v, st_v, sembt_v, sem, semw):
        wid = lax.axis_index("s") * NC + lax.axis_index("c")
        wbase = wid * SPW
        pltpu.sync_copy(len_ref.at[pl.ds(wbase, SPW)], lens_v)
        iota16 = lax.broadcasted_iota(jnp.int32, (NL,), 0)
        rowsel = iota16 * L
        dsel = iota16 * D

        def chunk_body(ci, carry):
            s0 = wbase + ci * C
            bt = s0 // 128
            bs0 = s0 % 128
            pltpu.sync_copy(text_ref.at[pl.ds(s0 * L, CL)], idx_v)
            cps = [
                pltpu.async_copy(
                    table_ref.at[idx_v.at[pl.ds(o, n)]],
                    rows_v.at[pl.ds(o, n)],
                    sem,
                )
                for (o, n) in pieces
            ]
            for cp in cps:
                cp.wait()
            @plsc.parallel_loop(0, L, unroll=4)
            def _tr(l):
                ridx = rowsel + l
                for d in range(D):
                    col = jnp.full((NL,), d, jnp.int32)
                    v = plsc.load_gather(rows_v, [ridx, col])
                    t5_v[l, d // 8, d % 8, :] = v
            wcp = pltpu.async_copy(
                t5_v, embs_ref.at[:, :, bt, :, pl.ds(bs0, C)], semw
            )
            lens16 = lens_v[pl.ds(ci * C, C)]
            for j in range(C):
                lenj = lens16[j]
                lenf = lenj.astype(jnp.float32)
                rb = j * L

                def ibody(i, accs):
                    r = rb + i
                    return tuple(
                        accs[g] + rows_v[r, pl.ds(g * NL, NL)]
                        for g in range(DG)
                    )

                z = jnp.zeros((NL,), jnp.float32)
                accs = lax.fori_loop(0, lenj, ibody, (z,) * DG)
                for g in range(DG):
                    st_v[pl.ds(j * D + g * NL, NL)] = accs[g] / lenf
            for d in range(D):
                v = plsc.load_gather(st_v, [dsel + d])
                sembt_v[d // 8, d % 8, :] = v
            pltpu.sync_copy(sembt_v, semb_ref.at[:, bt, :, pl.ds(bs0, C)])
            wcp.wait()
            return carry

        lax.fori_loop(0, NCH, chunk_body, 0)

    return sc_kernel


def kernel(text, text_len, table):
    B, L = text.shape
    V, D = table.shape
    sc = _build_kernel(B, L, V, D)
    embs5, semb4 = sc(text.reshape(B * L), text_len, table)
    embs = embs5.transpose(2, 4, 0, 1, 3).reshape(B, L, D)
    semb = semb4.transpose(1, 3, 0, 2).reshape(B, D)
    return embs, semb


# speedup vs baseline: 1.2635x; 1.2635x over previous
import functools

import jax
import jax.numpy as jnp
from jax import lax
from jax.experimental import pallas as pl
from jax.experimental.pallas import tpu as pltpu
from jax.experimental.pallas import tpu_sc as plsc


def _build_kernel(B, L, V, D):
    info = plsc.get_sparse_core_info()
    NC, NS, NL = info.num_cores, info.num_subcores, info.num_lanes
    NW = NC * NS
    assert B % NW == 0
    SPW = B // NW
    C = 16
    assert SPW % C == 0
    NCH = SPW // C
    CL = C * L
    assert D % NL == 0
    DG = D // NL
    pieces = []
    off = 0
    while off < CL:
        n = min(128, CL - off)
        pieces.append((off, n))
        off += n

    mesh = plsc.VectorSubcoreMesh(core_axis_name="c", subcore_axis_name="s")

    @functools.partial(
        pl.kernel,
        mesh=mesh,
        compiler_params=pltpu.CompilerParams(use_tc_tiling_on_sc=False),
        out_type=(
            jax.ShapeDtypeStruct((B * L, D), jnp.float32),
            jax.ShapeDtypeStruct((B, D), jnp.float32),
        ),
        scratch_types=[
            pltpu.VMEM((CL,), jnp.int32),
            pltpu.VMEM((CL, 2 * D), jnp.float32),
            pltpu.VMEM((SPW,), jnp.int32),
            pltpu.VMEM((C, D), jnp.float32),
            pltpu.SemaphoreType.DMA,
        ],
    )
    def sc_kernel(text_ref, len_ref, table_ref, embs_ref, semb_ref,
                  idx_v, rows_v, lens_v, semb_v, sem):
        wid = lax.axis_index("s") * NC + lax.axis_index("c")
        wbase = wid * SPW
        pltpu.sync_copy(len_ref.at[pl.ds(wbase, SPW)], lens_v)
        lane = lax.broadcasted_iota(jnp.int32, (NL,), 0)

        def chunk_body(ci, carry):
            s0 = wbase + ci * C
            pltpu.sync_copy(text_ref.at[pl.ds(s0 * L, CL)], idx_v)
            cps = [
                pltpu.async_copy(
                    table_ref.at[idx_v.at[pl.ds(o, n)]],
                    rows_v.at[pl.ds(o, n)],
                    sem,
                )
                for (o, n) in pieces
            ]
            for cp in cps:
                cp.wait()
            pltpu.sync_copy(
                rows_v.at[pl.ds(0, CL), pl.ds(0, D)],
                embs_ref.at[pl.ds(s0 * L, CL)],
            )
            lens16 = lens_v[pl.ds(ci * C, C)]
            for j in range(C):
                lenj = lens16[j]
                lenf = lenj.astype(jnp.float32)
                rb = j * L

                def ibody(i, accs):
                    r = rb + i
                    return tuple(
                        accs[g] + rows_v[r, pl.ds(g * NL, NL)]
                        for g in range(DG)
                    )

                z = jnp.zeros((NL,), jnp.float32)
                accs = lax.fori_loop(0, lenj, ibody, (z,) * DG)
                for g in range(DG):
                    semb_v[j, pl.ds(g * NL, NL)] = accs[g] / lenf
            pltpu.sync_copy(semb_v, semb_ref.at[pl.ds(s0, C)])
            return carry

        lax.fori_loop(0, NCH, chunk_body, 0)

    return sc_kernel


def kernel(text, text_len, table):
    B, L = text.shape
    V, D = table.shape
    sc = _build_kernel(B, L, V, D)
    pad = jnp.zeros((V, D), jnp.float32)
    tab128 = jnp.concatenate([table, pad], axis=1)
    embs_flat, semb = sc(text.reshape(B * L), text_len, tab128)
    return embs_flat.reshape(B, L, D), semb


# speedup vs baseline: 1.3485x; 1.0673x over previous
import functools

import jax
import jax.numpy as jnp
from jax import lax
from jax.experimental import pallas as pl
from jax.experimental.pallas import tpu as pltpu
from jax.experimental.pallas import tpu_sc as plsc


def _build_kernel(B, L, V, D):
    info = plsc.get_sparse_core_info()
    NC, NS, NL = info.num_cores, info.num_subcores, info.num_lanes
    NW = NC * NS
    assert B % NW == 0
    SPW = B // NW
    C = 32
    assert SPW % C == 0
    NCH = SPW // C
    CL = C * L
    assert D % NL == 0
    DG = D // NL
    pieces = []
    off = 0
    while off < CL:
        n = min(128, CL - off)
        pieces.append((off, n))
        off += n

    mesh = plsc.VectorSubcoreMesh(core_axis_name="c", subcore_axis_name="s")

    @functools.partial(
        pl.kernel,
        mesh=mesh,
        compiler_params=pltpu.CompilerParams(use_tc_tiling_on_sc=False),
        out_type=(
            jax.ShapeDtypeStruct((B * L, D), jnp.float32),
            jax.ShapeDtypeStruct((B, D), jnp.float32),
        ),
        scratch_types=[
            pltpu.VMEM((CL,), jnp.int32),
            pltpu.VMEM((CL, D), jnp.float32),
            pltpu.VMEM((SPW,), jnp.int32),
            pltpu.VMEM((C, D), jnp.float32),
            pltpu.SemaphoreType.DMA,
        ],
    )
    def sc_kernel(text_ref, len_ref, table_ref, embs_ref, semb_ref,
                  idx_v, rows_v, lens_v, semb_v, sem):
        wid = lax.axis_index("s") * NC + lax.axis_index("c")
        wbase = wid * SPW
        pltpu.sync_copy(len_ref.at[pl.ds(wbase, SPW)], lens_v)
        lane = lax.broadcasted_iota(jnp.int32, (NL,), 0)

        def chunk_body(ci, carry):
            s0 = wbase + ci * C
            pltpu.sync_copy(text_ref.at[pl.ds(s0 * L, CL)], idx_v)
            cps = [
                pltpu.async_copy(
                    table_ref.at[idx_v.at[pl.ds(o, n)]],
                    rows_v.at[pl.ds(o, n)],
                    sem,
                )
                for (o, n) in pieces
            ]
            for cp in cps:
                cp.wait()
            pltpu.sync_copy(rows_v, embs_ref.at[pl.ds(s0 * L, CL)])
            lens16 = lens_v[pl.ds(ci * C, C)]
            for j in range(C):
                lenj = lens16[j]
                lenf = lenj.astype(jnp.float32)
                rb = j * L

                def ibody(i, accs):
                    r = rb + i
                    return tuple(
                        accs[g] + rows_v[r, pl.ds(g * NL, NL)]
                        for g in range(DG)
                    )

                z = jnp.zeros((NL,), jnp.float32)
                accs = lax.fori_loop(0, lenj, ibody, (z,) * DG)
                for g in range(DG):
                    semb_v[j, pl.ds(g * NL, NL)] = accs[g] / lenf
            pltpu.sync_copy(semb_v, semb_ref.at[pl.ds(s0, C)])
            return carry

        lax.fori_loop(0, NCH, chunk_body, 0)

    return sc_kernel


def kernel(text, text_len, table):
    B, L = text.shape
    V, D = table.shape
    sc = _build_kernel(B, L, V, D)
    embs_flat, semb = sc(text.reshape(B * L), text_len, table)
    return embs_flat.reshape(B, L, D), semb


# speedup vs baseline: 1.4087x; 1.0446x over previous
import functools

import jax
import jax.numpy as jnp
from jax import lax
from jax.experimental import pallas as pl
from jax.experimental.pallas import tpu as pltpu
from jax.experimental.pallas import tpu_sc as plsc


def _build_kernel(B, L, V, D):
    info = plsc.get_sparse_core_info()
    NC, NS, NL = info.num_cores, info.num_subcores, info.num_lanes
    NW = NC * NS
    assert B % NW == 0
    SPW = B // NW
    C = 32
    assert SPW % C == 0
    NCH = SPW // C
    CL = C * L
    assert D % NL == 0
    DG = D // NL
    pieces = []
    off = 0
    while off < CL:
        n = min(128, CL - off)
        pieces.append((off, n))
        off += n

    mesh = plsc.VectorSubcoreMesh(core_axis_name="c", subcore_axis_name="s")

    @functools.partial(
        pl.kernel,
        mesh=mesh,
        compiler_params=pltpu.CompilerParams(use_tc_tiling_on_sc=False),
        out_type=(
            jax.ShapeDtypeStruct((B * L, D), jnp.float32),
            jax.ShapeDtypeStruct((B, D), jnp.float32),
        ),
        scratch_types=[
            pltpu.VMEM((CL,), jnp.int32),
            pltpu.VMEM((CL, D), jnp.float32),
            pltpu.VMEM((SPW,), jnp.int32),
            pltpu.VMEM((C, D), jnp.float32),
            pltpu.SemaphoreType.DMA,
            pltpu.SemaphoreType.DMA,
        ],
    )
    def sc_kernel(text_ref, len_ref, table_ref, embs_ref, semb_ref,
                  idx_v, rows_v, lens_v, semb_v, sem, semw):
        wid = lax.axis_index("s") * NC + lax.axis_index("c")
        wbase = wid * SPW
        pltpu.sync_copy(len_ref.at[pl.ds(wbase, SPW)], lens_v)
        lane = lax.broadcasted_iota(jnp.int32, (NL,), 0)

        def chunk_body(ci, carry):
            s0 = wbase + ci * C
            pltpu.sync_copy(text_ref.at[pl.ds(s0 * L, CL)], idx_v)
            cps = [
                pltpu.async_copy(
                    table_ref.at[idx_v.at[pl.ds(o, n)]],
                    rows_v.at[pl.ds(o, n)],
                    sem,
                )
                for (o, n) in pieces
            ]
            for cp in cps:
                cp.wait()
            wcp = pltpu.async_copy(rows_v, embs_ref.at[pl.ds(s0 * L, CL)], semw)
            lens16 = lens_v[pl.ds(ci * C, C)]
            for j in range(C):
                lenj = lens16[j]
                lenf = lenj.astype(jnp.float32)
                rb = j * L

                def ibody(i, accs):
                    r = rb + i
                    return tuple(
                        accs[g] + rows_v[r, pl.ds(g * NL, NL)]
                        for g in range(DG)
                    )

                z = jnp.zeros((NL,), jnp.float32)
                accs = lax.fori_loop(0, lenj, ibody, (z,) * DG)
                for g in range(DG):
                    semb_v[j, pl.ds(g * NL, NL)] = accs[g] / lenf
            pltpu.sync_copy(semb_v, semb_ref.at[pl.ds(s0, C)])
            wcp.wait()
            return carry

        lax.fori_loop(0, NCH, chunk_body, 0)

    return sc_kernel


def kernel(text, text_len, table):
    B, L = text.shape
    V, D = table.shape
    sc = _build_kernel(B, L, V, D)
    embs_flat, semb = sc(text.reshape(B * L), text_len, table)
    return embs_flat.reshape(B, L, D), semb


# speedup vs baseline: 1.4141x; 1.0038x over previous
import functools

import jax
import jax.numpy as jnp
from jax import lax
from jax.experimental import pallas as pl
from jax.experimental.pallas import tpu as pltpu
from jax.experimental.pallas import tpu_sc as plsc


def _build_kernel(B, L, V, D):
    info = plsc.get_sparse_core_info()
    NC, NS, NL = info.num_cores, info.num_subcores, info.num_lanes
    NW = NC * NS
    assert B % NW == 0
    SPW = B // NW
    C = 32
    assert SPW % C == 0
    NCH = SPW // C
    CL = C * L
    assert D % NL == 0
    DG = D // NL
    PS = 80
    assert CL % (2 * PS) == 0
    pieces = [(o, PS) for o in range(0, CL, PS)]
    HALF = len(pieces) // 2
    HCL = CL // 2
    HC = C // 2

    mesh = plsc.VectorSubcoreMesh(core_axis_name="c", subcore_axis_name="s")

    @functools.partial(
        pl.kernel,
        mesh=mesh,
        compiler_params=pltpu.CompilerParams(use_tc_tiling_on_sc=False),
        out_type=(
            jax.ShapeDtypeStruct((B * L, D), jnp.float32),
            jax.ShapeDtypeStruct((B, D), jnp.float32),
        ),
        scratch_types=[
            pltpu.VMEM((CL,), jnp.int32),
            pltpu.VMEM((CL, D), jnp.float32),
            pltpu.VMEM((SPW,), jnp.int32),
            pltpu.VMEM((C, D), jnp.float32),
            pltpu.SemaphoreType.DMA,
            pltpu.SemaphoreType.DMA,
        ],
    )
    def sc_kernel(text_ref, len_ref, table_ref, embs_ref, semb_ref,
                  idx_v, rows_v, lens_v, semb_v, sem, semw):
        wid = lax.axis_index("s") * NC + lax.axis_index("c")
        wbase = wid * SPW
        pltpu.sync_copy(len_ref.at[pl.ds(wbase, SPW)], lens_v)
        lane = lax.broadcasted_iota(jnp.int32, (NL,), 0)

        def chunk_body(ci, carry):
            s0 = wbase + ci * C

            def compute_half(h):
                for j in range(h * HC, (h + 1) * HC):
                    lenj = lens16[j]
                    lenf = lenj.astype(jnp.float32)
                    rb = j * L

                    def ibody(i, accs):
                        r = rb + i
                        return tuple(
                            accs[g] + rows_v[r, pl.ds(g * NL, NL)]
                            for g in range(DG)
                        )

                    z = jnp.zeros((NL,), jnp.float32)
                    accs = lax.fori_loop(0, lenj, ibody, (z,) * DG)
                    for g in range(DG):
                        semb_v[j, pl.ds(g * NL, NL)] = accs[g] / lenf
            pltpu.sync_copy(text_ref.at[pl.ds(s0 * L, CL)], idx_v)
            cps = [
                pltpu.async_copy(
                    table_ref.at[idx_v.at[pl.ds(o, n)]],
                    rows_v.at[pl.ds(o, n)],
                    sem,
                )
                for (o, n) in pieces
            ]
            lens16 = lens_v[pl.ds(ci * C, C)]
            wcps = []
            for h in range(2):
                for cp in cps[h * HALF:(h + 1) * HALF]:
                    cp.wait()
                wcps.append(pltpu.async_copy(
                    rows_v.at[pl.ds(h * HCL, HCL)],
                    embs_ref.at[pl.ds(s0 * L + h * HCL, HCL)],
                    semw,
                ))
                compute_half(h)
            for wcp in wcps:
                wcp.wait()
            pltpu.sync_copy(semb_v, semb_ref.at[pl.ds(s0, C)])
            return carry

        lax.fori_loop(0, NCH, chunk_body, 0)

    return sc_kernel


def kernel(text, text_len, table):
    B, L = text.shape
    V, D = table.shape
    sc = _build_kernel(B, L, V, D)
    embs_flat, semb = sc(text.reshape(B * L), text_len, table)
    return embs_flat.reshape(B, L, D), semb


# speedup vs baseline: 1.4147x; 1.0004x over previous
import functools

import jax
import jax.numpy as jnp
from jax import lax
from jax.experimental import pallas as pl
from jax.experimental.pallas import tpu as pltpu
from jax.experimental.pallas import tpu_sc as plsc


def _build_kernel(B, L, V, D):
    info = plsc.get_sparse_core_info()
    NC, NS, NL = info.num_cores, info.num_subcores, info.num_lanes
    NW = NC * NS
    assert B % NW == 0
    SPW = B // NW
    C = 32
    assert SPW % C == 0
    NCH = SPW // C
    CL = C * L
    assert D % NL == 0
    DG = D // NL
    PS = 80
    assert CL % (2 * PS) == 0
    pieces = [(o, PS) for o in range(0, CL, PS)]
    HALF = len(pieces) // 2
    HCL = CL // 2
    HC = C // 2

    mesh = plsc.VectorSubcoreMesh(core_axis_name="c", subcore_axis_name="s")

    @functools.partial(
        pl.kernel,
        mesh=mesh,
        compiler_params=pltpu.CompilerParams(use_tc_tiling_on_sc=False),
        out_type=(
            jax.ShapeDtypeStruct((B * L, D), jnp.float32),
            jax.ShapeDtypeStruct((B, D), jnp.float32),
        ),
        scratch_types=[
            pltpu.VMEM((CL,), jnp.int32),
            pltpu.VMEM((CL, D), jnp.float32),
            pltpu.VMEM((SPW,), jnp.int32),
            pltpu.VMEM((C, D), jnp.float32),
            pltpu.SemaphoreType.DMA,
            pltpu.SemaphoreType.DMA,
        ],
    )
    def sc_kernel(text_ref, len_ref, table_ref, embs_ref, semb_ref,
                  idx_v, rows_v, lens_v, semb_v, sem, semw):
        wid = lax.axis_index("s") * NC + lax.axis_index("c")
        wbase = wid * SPW
        pltpu.sync_copy(len_ref.at[pl.ds(wbase, SPW)], lens_v)

        def chunk_body(ci, carry):
            s0 = wbase + ci * C

            def compute_half(h):
                for j in range(h * HC, (h + 1) * HC):
                    lenj = lens16[j]
                    lenf = lenj.astype(jnp.float32)
                    rb = j * L

                    def ibody(i, accs):
                        r = rb + i
                        return tuple(
                            accs[g] + rows_v[r, pl.ds(g * NL, NL)]
                            for g in range(DG)
                        )

                    z = jnp.zeros((NL,), jnp.float32)
                    accs = lax.fori_loop(0, lenj, ibody, (z,) * DG)
                    for g in range(DG):
                        semb_v[j, pl.ds(g * NL, NL)] = accs[g] / lenf
            pltpu.sync_copy(text_ref.at[pl.ds(s0 * L, CL)], idx_v)
            cps = [
                pltpu.async_copy(
                    table_ref.at[idx_v.at[pl.ds(o, n)]],
                    rows_v.at[pl.ds(o, n)],
                    sem,
                )
                for (o, n) in pieces
            ]
            lens16 = lens_v[pl.ds(ci * C, C)]
            wcps = []
            for h in range(2):
                for cp in cps[h * HALF:(h + 1) * HALF]:
                    cp.wait()
                wcps.append(pltpu.async_copy(
                    rows_v.at[pl.ds(h * HCL, HCL)],
                    embs_ref.at[pl.ds(s0 * L + h * HCL, HCL)],
                    semw,
                ))
                compute_half(h)
            for wcp in wcps:
                wcp.wait()
            pltpu.sync_copy(semb_v, semb_ref.at[pl.ds(s0, C)])
            return carry

        lax.fori_loop(0, NCH, chunk_body, 0)

    return sc_kernel


def kernel(text, text_len, table):
    B, L = text.shape
    V, D = table.shape
    sc = _build_kernel(B, L, V, D)
    embs_flat, semb = sc(text.reshape(B * L), text_len, table)
    return embs_flat.reshape(B, L, D), semb
